# baseline (speedup 1.0000x reference)
import functools

import jax
import jax.numpy as jnp
from jax import lax
from jax.experimental import pallas as pl
from jax.experimental.pallas import tpu as pltpu
from jax.experimental.pallas import tpu_sc as plsc

NC = 2
NS = 16
NW = NC * NS
L = 16


@functools.partial(jax.jit, static_argnums=(6, 7))
def _run(ui2, pi2, ni2, gamma_users, gamma_items, beta_items, dim, bpw):
    nch = ui2.shape[0] // NW
    chb = ui2.shape[1]
    ng = bpw // L
    mesh = plsc.VectorSubcoreMesh(
        core_axis_name="c", subcore_axis_name="s",
        num_cores=NC, num_subcores=NS)

    @functools.partial(
        pl.kernel,
        out_type=jax.ShapeDtypeStruct((NW * bpw,), jnp.float32),
        mesh=mesh,
        scratch_types=[
            pltpu.VMEM((nch, chb), jnp.int32),
            pltpu.VMEM((nch, chb), jnp.int32),
            pltpu.VMEM((nch, chb), jnp.int32),
            pltpu.VMEM((bpw, dim), jnp.float32),
            pltpu.VMEM((bpw, dim), jnp.float32),
            pltpu.VMEM((bpw, dim), jnp.float32),
            pltpu.VMEM((bpw,), jnp.float32),
            pltpu.VMEM((bpw,), jnp.float32),
            pltpu.VMEM((bpw,), jnp.float32),
            pltpu.SemaphoreType.DMA,
        ],
        compiler_params=pltpu.CompilerParams(
            needs_layout_passes=False, use_tc_tiling_on_sc=False),
    )
    def k(ui_hbm, pi_hbm, ni_hbm, gu_hbm, gi_hbm, bb_hbm, out_hbm,
          ui_v, pi_v, ni_v, urows, prows, nrows, pb_v, nb_v, out_v, sem):
        wid = lax.axis_index("s") * NC + lax.axis_index("c")
        base = wid * bpw

        pltpu.sync_copy(ui_hbm.at[pl.ds(wid * nch, nch)], ui_v)
        pltpu.sync_copy(pi_hbm.at[pl.ds(wid * nch, nch)], pi_v)
        pltpu.sync_copy(ni_hbm.at[pl.ds(wid * nch, nch)], ni_v)

        cps = []
        for j in range(nch):
            rows = pl.ds(j * chb, chb)
            cps.append(pltpu.async_copy(gu_hbm.at[ui_v.at[j]], urows.at[rows], sem))
            cps.append(pltpu.async_copy(gi_hbm.at[pi_v.at[j]], prows.at[rows], sem))
            cps.append(pltpu.async_copy(gi_hbm.at[ni_v.at[j]], nrows.at[rows], sem))
            cps.append(pltpu.async_copy(bb_hbm.at[pi_v.at[j]], pb_v.at[rows], sem))
            cps.append(pltpu.async_copy(bb_hbm.at[ni_v.at[j]], nb_v.at[rows], sem))
        for c in cps:
            c.wait()

        zf = jnp.zeros((L,), jnp.float32)

        def group(g, carry):
            idx0 = g * L + lax.iota(jnp.int32, L)
            pb = pb_v[pl.ds(g * L, L)]
            nb = nb_v[pl.ds(g * L, L)]
            accs = [zf, zf, zf, zf]
            for d in range(dim):
                idxd = jnp.full((L,), d, jnp.int32)
                u = plsc.load_gather(urows, [idx0, idxd])
                p = plsc.load_gather(prows, [idx0, idxd])
                n = plsc.load_gather(nrows, [idx0, idxd])
                accs[d % 4] = accs[d % 4] + u * (p - n)
            res = (accs[0] + accs[1]) + (accs[2] + accs[3]) + pb - nb
            out_v[pl.ds(g * L, L)] = res
            return carry

        lax.fori_loop(0, ng, group, 0)
        pltpu.sync_copy(out_v, out_hbm.at[pl.ds(base, bpw)])

    return k(ui2, pi2, ni2, gamma_users, gamma_items, beta_items)


def kernel(ui, pi, ni, gamma_users, gamma_items, beta_items):
    b = ui.shape[0]
    dim = gamma_users.shape[1]
    bpw = b // NW
    chb = min(128, bpw)
    nch = bpw // chb
    ui2 = ui.astype(jnp.int32).reshape(NW * nch, chb)
    pi2 = pi.astype(jnp.int32).reshape(NW * nch, chb)
    ni2 = ni.astype(jnp.int32).reshape(NW * nch, chb)
    beta1d = beta_items.reshape(-1)
    out = _run(ui2, pi2, ni2, gamma_users, gamma_items, beta1d, dim, bpw)
    return out.reshape(b, 1, 1)


# speedup vs baseline: 1.0022x; 1.0022x over previous
import functools

import jax
import jax.numpy as jnp
from jax import lax
from jax.experimental import pallas as pl
from jax.experimental.pallas import tpu as pltpu
from jax.experimental.pallas import tpu_sc as plsc

NC = 2
NS = 16
NW = NC * NS
L = 16
CHB = 128


@functools.partial(jax.jit, static_argnums=(3,))
def _run_bias(pi2, ni2, beta1d, bpw):
    nch = pi2.shape[0] // NW
    mesh = plsc.VectorSubcoreMesh(
        core_axis_name="c", subcore_axis_name="s",
        num_cores=NC, num_subcores=NS)

    @functools.partial(
        pl.kernel,
        out_type=jax.ShapeDtypeStruct((NW * bpw,), jnp.float32),
        mesh=mesh,
        scratch_types=[
            pltpu.VMEM((nch, CHB), jnp.int32),
            pltpu.VMEM((nch, CHB), jnp.int32),
            pltpu.VMEM((bpw,), jnp.float32),
            pltpu.VMEM((bpw,), jnp.float32),
            pltpu.VMEM((bpw,), jnp.float32),
            pltpu.SemaphoreType.DMA,
        ],
        compiler_params=pltpu.CompilerParams(
            needs_layout_passes=False, use_tc_tiling_on_sc=False),
    )
    def k(pi_hbm, ni_hbm, bb_hbm, out_hbm, pi_v, ni_v, pb_v, nb_v, out_v,
          sem):
        wid = lax.axis_index("s") * NC + lax.axis_index("c")
        pltpu.sync_copy(pi_hbm.at[pl.ds(wid * nch, nch)], pi_v)
        pltpu.sync_copy(ni_hbm.at[pl.ds(wid * nch, nch)], ni_v)
        cps = []
        for j in range(nch):
            rows = pl.ds(j * CHB, CHB)
            cps.append(pltpu.async_copy(
                bb_hbm.at[pi_v.at[j]], pb_v.at[rows], sem))
            cps.append(pltpu.async_copy(
                bb_hbm.at[ni_v.at[j]], nb_v.at[rows], sem))
        for c in cps:
            c.wait()

        def group(g, carry):
            sl = pl.ds(g * L, L)
            out_v[sl] = pb_v[sl] - nb_v[sl]
            return carry

        lax.fori_loop(0, bpw // L, group, 0)
        pltpu.sync_copy(out_v, out_hbm.at[pl.ds(wid * bpw, bpw)])

    return k(pi2, ni2, beta1d)


@functools.partial(jax.jit, static_argnums=(7, 8))
def _run_main(ui2, pi2, ni2, gu2, gi2, bdiff, dummy, dim, bpw):
    del dummy
    nch = ui2.shape[0] // NW
    npass = (bpw + 255) // 256
    pb = bpw // npass
    pch = pb // CHB
    ngrp = pb // L
    mesh = plsc.VectorSubcoreMesh(
        core_axis_name="c", subcore_axis_name="s",
        num_cores=NC, num_subcores=NS)

    @functools.partial(
        pl.kernel,
        out_type=jax.ShapeDtypeStruct((NW * bpw,), jnp.float32),
        mesh=mesh,
        scratch_types=[
            pltpu.VMEM((nch, CHB), jnp.int32),
            pltpu.VMEM((nch, CHB), jnp.int32),
            pltpu.VMEM((nch, CHB), jnp.int32),
            pltpu.VMEM((pch, CHB), jnp.int32),
            pltpu.VMEM((pch, CHB), jnp.int32),
            pltpu.VMEM((pch, CHB), jnp.int32),
            pltpu.VMEM((bpw,), jnp.int32),
            pltpu.VMEM((bpw,), jnp.int32),
            pltpu.VMEM((bpw,), jnp.int32),
            pltpu.VMEM((pb, 2 * 64), jnp.float32),
            pltpu.VMEM((pb, 2 * 64), jnp.float32),
            pltpu.VMEM((pb, 2 * 64), jnp.float32),
            pltpu.VMEM((bpw,), jnp.float32),
            pltpu.VMEM((bpw,), jnp.float32),
            pltpu.SemaphoreType.DMA,
        ],
        compiler_params=pltpu.CompilerParams(needs_layout_passes=False),
    )
    def k(ui_hbm, pi_hbm, ni_hbm, gu_hbm, gi_hbm, bd_hbm, out_hbm,
          ui_v, pi_v, ni_v, ju, jp, jn, pau, pap, pan, urows, prows, nrows,
          bd_v, out_v, sem):
        wid = lax.axis_index("s") * NC + lax.axis_index("c")
        base = wid * bpw
        pltpu.sync_copy(ui_hbm.at[pl.ds(wid * nch, nch)], ui_v)
        pltpu.sync_copy(pi_hbm.at[pl.ds(wid * nch, nch)], pi_v)
        pltpu.sync_copy(ni_hbm.at[pl.ds(wid * nch, nch)], ni_v)
        pltpu.sync_copy(bd_hbm.at[pl.ds(base, bpw)], bd_v)

        lanes = lax.iota(jnp.int32, L)
        zf = jnp.zeros((L,), jnp.float32)

        for p in range(npass):
            for c in range(pch):
                src = p * pch + c
                for t in range(CHB // L):
                    sl = pl.ds(t * L, L)
                    gsl = pl.ds(src * CHB + t * L, L)
                    ru = ui_v.at[src][sl]
                    rp = pi_v.at[src][sl]
                    rn = ni_v.at[src][sl]
                    ju.at[c][sl] = ru >> 1
                    jp.at[c][sl] = rp >> 1
                    jn.at[c][sl] = rn >> 1
                    pau[gsl] = (ru & 1) << 6
                    pap[gsl] = (rp & 1) << 6
                    pan[gsl] = (rn & 1) << 6
            cps = []
            for c in range(pch):
                rows = pl.ds(c * CHB, CHB)
                cps.append(pltpu.async_copy(
                    gu_hbm.at[ju.at[c]], urows.at[rows], sem))
                cps.append(pltpu.async_copy(
                    gi_hbm.at[jp.at[c]], prows.at[rows], sem))
                cps.append(pltpu.async_copy(
                    gi_hbm.at[jn.at[c]], nrows.at[rows], sem))
            for c in cps:
                c.wait()

            def group(g, carry):
                loc = pl.ds(g * L, L)
                glb = pl.ds(p * pb + g * L, L)
                pu = pau[glb]
                pp = pap[glb]
                pn = pan[glb]
                lidx = g * L + lanes
                accs = [zf, zf, zf, zf]
                for d in range(dim):
                    u = plsc.load_gather(urows, [lidx, pu + d])
                    pr = plsc.load_gather(prows, [lidx, pp + d])
                    n = plsc.load_gather(nrows, [lidx, pn + d])
                    accs[d % 4] = accs[d % 4] + u * (pr - n)
                res = (accs[0] + accs[1]) + (accs[2] + accs[3]) + bd_v[glb]
                out_v[glb] = res
                return carry

            lax.fori_loop(0, ngrp, group, 0)

        pltpu.sync_copy(out_v, out_hbm.at[pl.ds(base, bpw)])

    return k(ui2, pi2, ni2, gu2, gi2, bdiff)


def kernel(ui, pi, ni, gamma_users, gamma_items, beta_items):
    b = ui.shape[0]
    rows, dim = gamma_users.shape
    bpw = b // NW
    nch = bpw // CHB
    ui2 = ui.astype(jnp.int32).reshape(NW * nch, CHB)
    pi2 = pi.astype(jnp.int32).reshape(NW * nch, CHB)
    ni2 = ni.astype(jnp.int32).reshape(NW * nch, CHB)
    gu2 = gamma_users.reshape(rows // 2, 2 * dim)
    gi2 = gamma_items.reshape(rows // 2, 2 * dim)
    beta1d = beta_items.reshape(-1)
    bdiff = _run_bias(pi2, ni2, beta1d, bpw)
    out = _run_main(ui2, pi2, ni2, gu2, gi2, bdiff, None, dim, bpw)
    return out.reshape(b, 1, 1)
